# initial kernel scaffold (unmeasured)
import jax
import jax.numpy as jnp
from jax import lax
from jax.experimental import pallas as pl
from jax.experimental.pallas import tpu as pltpu

N_DEV = 8
N_STEPS = 3


def kernel(x, W1, W2):
    m, k = x.shape
    _, h_per = W1.shape
    _, n = W2.shape

    def body(x_ref, w1_ref, w2_ref, out_ref, acc_ref, comm_ref,
             send_sems, recv_sems):
        me = lax.axis_index("i")

        h = jnp.maximum(
            jnp.dot(x_ref[...], w1_ref[...],
                    preferred_element_type=jnp.float32),
            0.0,
        )
        acc_ref[...] = jnp.dot(h, w2_ref[...],
                               preferred_element_type=jnp.float32)

        for s in range(N_STEPS):
            partner = me ^ (1 << s)
            rdma = pltpu.make_async_remote_copy(
                src_ref=acc_ref,
                dst_ref=comm_ref.at[s],
                send_sem=send_sems.at[s],
                recv_sem=recv_sems.at[s],
                device_id=(partner,),
                device_id_type=pl.DeviceIdType.MESH,
            )
            rdma.start()
            rdma.wait()
            acc_ref[...] = acc_ref[...] + comm_ref[s]

        out_ref[...] = acc_ref[...]

    return pl.pallas_call(
        body,
        out_shape=jax.ShapeDtypeStruct((m, n), jnp.float32),
        in_specs=[pl.BlockSpec(memory_space=pltpu.VMEM)] * 3,
        out_specs=pl.BlockSpec(memory_space=pltpu.VMEM),
        scratch_shapes=[
            pltpu.VMEM((m, n), jnp.float32),
            pltpu.VMEM((N_STEPS, m, n), jnp.float32),
            pltpu.SemaphoreType.DMA((N_STEPS,)),
            pltpu.SemaphoreType.DMA((N_STEPS,)),
        ],
        compiler_params=pltpu.CompilerParams(collective_id=0),
    )(x, W1, W2)


# baseline (device time: 54017 ns/iter reference)
import jax
import jax.numpy as jnp
from jax import lax
from jax.experimental import pallas as pl
from jax.experimental.pallas import tpu as pltpu

N_DEV = 8
N_STEPS = 3


def kernel(x, W1, W2):
    m, k = x.shape
    _, h_per = W1.shape
    _, n = W2.shape

    def body(x_ref, w1_ref, w2_ref, out_ref, acc_ref, comm_ref,
             send_sems, recv_sems):
        me = lax.axis_index("i")

        h = jnp.maximum(
            jnp.dot(x_ref[...], w1_ref[...],
                    preferred_element_type=jnp.float32),
            0.0,
        )
        acc_ref[...] = jnp.dot(h, w2_ref[...],
                               preferred_element_type=jnp.float32)

        for s in range(N_STEPS):
            partner = me ^ (1 << s)
            rdma = pltpu.make_async_remote_copy(
                src_ref=acc_ref,
                dst_ref=comm_ref.at[s],
                send_sem=send_sems.at[s],
                recv_sem=recv_sems.at[s],
                device_id=(partner,),
                device_id_type=pl.DeviceIdType.MESH,
            )
            rdma.start()
            rdma.wait()
            acc_ref[...] = acc_ref[...] + comm_ref[s]

        out_ref[...] = acc_ref[...]

    return pl.pallas_call(
        body,
        out_shape=jax.ShapeDtypeStruct((m, n), jnp.float32),
        in_specs=[pl.BlockSpec(memory_space=pltpu.VMEM)] * 3,
        out_specs=pl.BlockSpec(memory_space=pltpu.VMEM),
        scratch_shapes=[
            pltpu.VMEM((m, n), jnp.float32),
            pltpu.VMEM((N_STEPS, m, n), jnp.float32),
            pltpu.SemaphoreType.DMA((N_STEPS,)),
            pltpu.SemaphoreType.DMA((N_STEPS,)),
        ],
    )(x, W1, W2)


# device time: 42792 ns/iter; 1.2623x vs baseline; 1.2623x over previous
import jax
import jax.numpy as jnp
from jax import lax
from jax.experimental import pallas as pl
from jax.experimental.pallas import tpu as pltpu

N_DEV = 8
MASKS = (1, 3, 4)
CONTRIBS = (256, 128, 64)


def kernel(x, W1, W2):
    m, k = x.shape
    _, h_per = W1.shape
    _, n = W2.shape
    assert m == 512 and n == 512

    def body(x_ref, w1_ref, w2_ref, out_ref, acc_ref, comm_ref,
             send_sems, recv_sems):
        me = lax.axis_index("i")
        sels = ((me ^ (me >> 1)) & 1, (me >> 1) & 1, (me >> 2) & 1)

        h = jnp.maximum(
            jnp.dot(x_ref[...], w1_ref[...],
                    preferred_element_type=jnp.float32),
            0.0,
        )
        acc_ref[...] = jnp.dot(h, w2_ref[...],
                               preferred_element_type=jnp.float32)

        rdmas = []

        comm_offs = (0, 256, 384)
        off = 0
        half = m // 2
        for s in range(3):
            partner = me ^ MASKS[s]
            keep_off = pl.multiple_of(off + sels[s] * half, 64)
            send_off = pl.multiple_of(off + (1 - sels[s]) * half, 64)
            rdma = pltpu.make_async_remote_copy(
                src_ref=acc_ref.at[pl.ds(send_off, half)],
                dst_ref=comm_ref.at[pl.ds(comm_offs[s], half)],
                send_sem=send_sems.at[s],
                recv_sem=recv_sems.at[s],
                device_id=(partner,),
                device_id_type=pl.DeviceIdType.MESH,
            )
            rdma.start()
            rdma.wait_recv()
            acc_ref[pl.ds(keep_off, half)] = (
                acc_ref[pl.ds(keep_off, half)]
                + comm_ref[pl.ds(comm_offs[s], half)]
            )
            rdmas.append(rdma)
            off = keep_off
            half //= 2


        cur = m // N_DEV
        for i, s in enumerate((2, 1, 0)):
            partner = me ^ MASKS[s]
            off = pl.multiple_of(off, 64)
            rdma = pltpu.make_async_remote_copy(
                src_ref=acc_ref.at[pl.ds(off, cur)],
                dst_ref=acc_ref.at[pl.ds(off, cur)],
                send_sem=send_sems.at[3 + i],
                recv_sem=recv_sems.at[3 + i],
                device_id=(partner,),
                device_id_type=pl.DeviceIdType.MESH,
            )
            rdma.start()
            rdma.wait_recv()
            rdmas.append(rdma)
            off = jnp.minimum(off, off ^ CONTRIBS[s])
            cur *= 2

        for r in rdmas:
            r.wait_send()

        out_ref[...] = acc_ref[...]

    return pl.pallas_call(
        body,
        out_shape=jax.ShapeDtypeStruct((m, n), jnp.float32),
        in_specs=[pl.BlockSpec(memory_space=pltpu.VMEM)] * 3,
        out_specs=pl.BlockSpec(memory_space=pltpu.VMEM),
        scratch_shapes=[
            pltpu.VMEM((m, n), jnp.float32),
            pltpu.VMEM((448, n), jnp.float32),
            pltpu.SemaphoreType.DMA((6,)),
            pltpu.SemaphoreType.DMA((6,)),
        ],
    )(x, W1, W2)


# device time: 42406 ns/iter; 1.2738x vs baseline; 1.0091x over previous
import jax
import jax.numpy as jnp
from jax import lax
from jax.experimental import pallas as pl
from jax.experimental.pallas import tpu as pltpu

N_DEV = 8
MASKS = (1, 3, 4)
CONTRIBS = (256, 128, 64)


def kernel(x, W1, W2):
    m, k = x.shape
    _, h_per = W1.shape
    _, n = W2.shape
    assert m == 512 and n == 512

    def body(x_ref, w1_ref, w2_ref, out_ref, acc_ref, comm_ref,
             send_sems, recv_sems):
        me = lax.axis_index("i")
        sels = ((me ^ (me >> 1)) & 1, (me >> 1) & 1, (me >> 2) & 1)

        def partial_rows(row_off, nrows):
            xs = x_ref[pl.ds(row_off, nrows)]
            h = jnp.maximum(
                jnp.dot(xs, w1_ref[...],
                        preferred_element_type=jnp.float32),
                0.0,
            )
            return jnp.dot(h, w2_ref[...],
                           preferred_element_type=jnp.float32)

        rdmas = []

        keep0 = pl.multiple_of(sels[0] * 256, 256)
        send0 = pl.multiple_of((1 - sels[0]) * 256, 256)
        acc_ref[pl.ds(send0, 256)] = partial_rows(send0, 256)
        rdma0 = pltpu.make_async_remote_copy(
            src_ref=acc_ref.at[pl.ds(send0, 256)],
            dst_ref=comm_ref.at[pl.ds(0, 256)],
            send_sem=send_sems.at[0],
            recv_sem=recv_sems.at[0],
            device_id=(me ^ MASKS[0],),
            device_id_type=pl.DeviceIdType.MESH,
        )
        rdma0.start()
        acc_ref[pl.ds(keep0, 256)] = partial_rows(keep0, 256)
        rdma0.wait_recv()
        acc_ref[pl.ds(keep0, 256)] = (
            acc_ref[pl.ds(keep0, 256)] + comm_ref[pl.ds(0, 256)]
        )
        rdmas.append(rdma0)

        comm_offs = (0, 256, 384)
        off = keep0
        half = 128
        for s in (1, 2):
            partner = me ^ MASKS[s]
            keep_off = pl.multiple_of(off + sels[s] * half, 64)
            send_off = pl.multiple_of(off + (1 - sels[s]) * half, 64)
            rdma = pltpu.make_async_remote_copy(
                src_ref=acc_ref.at[pl.ds(send_off, half)],
                dst_ref=comm_ref.at[pl.ds(comm_offs[s], half)],
                send_sem=send_sems.at[s],
                recv_sem=recv_sems.at[s],
                device_id=(partner,),
                device_id_type=pl.DeviceIdType.MESH,
            )
            rdma.start()
            rdma.wait_recv()
            acc_ref[pl.ds(keep_off, half)] = (
                acc_ref[pl.ds(keep_off, half)]
                + comm_ref[pl.ds(comm_offs[s], half)]
            )
            rdmas.append(rdma)
            off = keep_off
            half //= 2

        out_ref[pl.ds(off, 64)] = acc_ref[pl.ds(off, 64)]

        cur = m // N_DEV
        for i, s in enumerate((2, 1, 0)):
            partner = me ^ MASKS[s]
            off = pl.multiple_of(off, 64)
            rdma = pltpu.make_async_remote_copy(
                src_ref=out_ref.at[pl.ds(off, cur)],
                dst_ref=out_ref.at[pl.ds(off, cur)],
                send_sem=send_sems.at[3 + i],
                recv_sem=recv_sems.at[3 + i],
                device_id=(partner,),
                device_id_type=pl.DeviceIdType.MESH,
            )
            rdma.start()
            rdma.wait_recv()
            rdmas.append(rdma)
            off = jnp.minimum(off, off ^ CONTRIBS[s])
            cur *= 2

        for r in rdmas:
            r.wait_send()

    return pl.pallas_call(
        body,
        out_shape=jax.ShapeDtypeStruct((m, n), jnp.float32),
        in_specs=[pl.BlockSpec(memory_space=pltpu.VMEM)] * 3,
        out_specs=pl.BlockSpec(memory_space=pltpu.VMEM),
        scratch_shapes=[
            pltpu.VMEM((m, n), jnp.float32),
            pltpu.VMEM((448, n), jnp.float32),
            pltpu.SemaphoreType.DMA((6,)),
            pltpu.SemaphoreType.DMA((6,)),
        ],
    )(x, W1, W2)


# device time: 18962 ns/iter; 2.8487x vs baseline; 2.2364x over previous
import jax
import jax.numpy as jnp
from jax import lax
from jax.experimental import pallas as pl
from jax.experimental.pallas import tpu as pltpu

N_DEV = 8
BLK = 64
NCOL = 256


def kernel(x, W1, W2):
    m, k = x.shape
    _, h_per = W1.shape
    _, n = W2.shape
    assert m == 512 and n == 512

    def body(x_ref, w1_ref, w2_ref, out_ref,
             acc0_ref, acc1_ref, stage0_ref, stage1_ref,
             comm0_ref, comm1_ref, ag_stage0_ref, ag_stage1_ref,
             ag_comm0_ref, ag_comm1_ref,
             rs_send_sems, rs_recv_sems, ag_send_sems, ag_recv_sems):
        me = lax.axis_index("i")
        my_off = pl.multiple_of(me * BLK, BLK)

        barrier_sem = pltpu.get_barrier_semaphore()
        pl.semaphore_signal(barrier_sem, inc=1)
        pl.semaphore_wait(barrier_sem, 1)

        accs = (acc0_ref, acc1_ref)
        stages = (stage0_ref, stage1_ref)
        comms = (comm0_ref, comm1_ref)
        ag_stages = (ag_stage0_ref, ag_stage1_ref)
        ag_comms = (ag_comm0_ref, ag_comm1_ref)

        h = jnp.maximum(
            jnp.dot(x_ref[...].astype(jnp.bfloat16),
                    w1_ref[...].astype(jnp.bfloat16),
                    preferred_element_type=jnp.float32),
            0.0,
        ).astype(jnp.bfloat16)

        rs_sends = []
        for c in (0, 1):
            part = jnp.dot(h, w2_ref[:, pl.ds(c * NCOL, NCOL)]
                           .astype(jnp.bfloat16),
                           preferred_element_type=jnp.float32)
            accs[c][...] = part
            stages[c][...] = part.astype(jnp.bfloat16)
            for j in range(N_DEV):
                rdma = pltpu.make_async_remote_copy(
                    src_ref=stages[c].at[pl.ds(j * BLK, BLK)],
                    dst_ref=comms[c].at[pl.ds(my_off, BLK)],
                    send_sem=rs_send_sems.at[c * N_DEV + j],
                    recv_sem=rs_recv_sems.at[c * N_DEV + me],
                    device_id=(j,),
                    device_id_type=pl.DeviceIdType.MESH,
                )
                pl.when(j != me)(rdma.start)
                rs_sends.append((j, rdma))

        ag = []
        for c in (0, 1):
            red = accs[c][pl.ds(my_off, BLK)]
            for mask in range(1, N_DEV):
                p = me ^ mask
                p_off = pl.multiple_of(p * BLK, BLK)
                recv = pltpu.make_async_remote_copy(
                    src_ref=stages[c].at[pl.ds(0, BLK)],
                    dst_ref=comms[c].at[pl.ds(p_off, BLK)],
                    send_sem=rs_send_sems.at[0],
                    recv_sem=rs_recv_sems.at[c * N_DEV + p],
                    device_id=(0,),
                    device_id_type=pl.DeviceIdType.MESH,
                )
                recv.wait_recv()
                red = red + comms[c][pl.ds(p_off, BLK)].astype(jnp.float32)
            out_ref[pl.ds(my_off, BLK), pl.ds(c * NCOL, NCOL)] = red
            ag_stages[c][...] = red.astype(jnp.bfloat16)
            for mask in range(1, N_DEV):
                p = me ^ mask
                rdma = pltpu.make_async_remote_copy(
                    src_ref=ag_stages[c],
                    dst_ref=ag_comms[c].at[pl.ds((mask - 1) * BLK, BLK)],
                    send_sem=ag_send_sems.at[c * (N_DEV - 1) + mask - 1],
                    recv_sem=ag_recv_sems.at[c * (N_DEV - 1) + mask - 1],
                    device_id=(p,),
                    device_id_type=pl.DeviceIdType.MESH,
                )
                rdma.start()
                ag.append(rdma)

        for c in (0, 1):
            for mask in range(1, N_DEV):
                p = me ^ mask
                ag[c * (N_DEV - 1) + mask - 1].wait_recv()
                out_ref[pl.ds(pl.multiple_of(p * BLK, BLK), BLK),
                        pl.ds(c * NCOL, NCOL)] = (
                    ag_comms[c][pl.ds((mask - 1) * BLK, BLK)]
                    .astype(jnp.float32)
                )

        for j, rdma in rs_sends:
            pl.when(j != me)(rdma.wait_send)
        for rdma in ag:
            rdma.wait_send()

    return pl.pallas_call(
        body,
        out_shape=jax.ShapeDtypeStruct((m, n), jnp.float32),
        in_specs=[pl.BlockSpec(memory_space=pltpu.VMEM)] * 3,
        out_specs=pl.BlockSpec(memory_space=pltpu.VMEM),
        compiler_params=pltpu.CompilerParams(collective_id=0),
        scratch_shapes=[
            pltpu.VMEM((m, NCOL), jnp.float32),
            pltpu.VMEM((m, NCOL), jnp.float32),
            pltpu.VMEM((m, NCOL), jnp.bfloat16),
            pltpu.VMEM((m, NCOL), jnp.bfloat16),
            pltpu.VMEM((m, NCOL), jnp.bfloat16),
            pltpu.VMEM((m, NCOL), jnp.bfloat16),
            pltpu.VMEM((BLK, NCOL), jnp.bfloat16),
            pltpu.VMEM((BLK, NCOL), jnp.bfloat16),
            pltpu.VMEM(((N_DEV - 1) * BLK, NCOL), jnp.bfloat16),
            pltpu.VMEM(((N_DEV - 1) * BLK, NCOL), jnp.bfloat16),
            pltpu.SemaphoreType.DMA((2 * N_DEV,)),
            pltpu.SemaphoreType.DMA((2 * N_DEV,)),
            pltpu.SemaphoreType.DMA((2 * (N_DEV - 1),)),
            pltpu.SemaphoreType.DMA((2 * (N_DEV - 1),)),
        ],
    )(x, W1, W2)
